# Initial kernel scaffold; baseline (speedup 1.0000x reference)
import jax
import jax.numpy as jnp
from jax.experimental import pallas as pl


def kernel(state, W1, b1, W2, b2, ln_g, ln_b, mem_keys, mem_values, V1, bv1, V2, bv2, V3, bv3):
    raise NotImplementedError("write your pallas kernel here")


# speedup vs baseline: 1.2847x; 1.2847x over previous
import functools

import jax
import jax.numpy as jnp
from jax.experimental import pallas as pl
from jax.experimental.pallas import tpu as pltpu

_M = 50000
_MP = 50176
_MC = _MP // 4
_BQ = 64
_K = 50


def _nec_kernel(state_ref, W1_ref, b1_ref, W2_ref, b2_ref, lng_ref, lnb_ref,
                mkT_ref, mv_ref, V1_ref, bv1_ref, V2_ref, bv2_ref, V3_ref,
                bv3_ref, out_ref, d2_ref):
    s = state_ref[...]

    h = jnp.maximum(
        jnp.dot(s, W1_ref[...], preferred_element_type=jnp.float32)
        + b1_ref[...], 0.0)
    k_raw = (jnp.dot(h, W2_ref[...], preferred_element_type=jnp.float32)
             + b2_ref[...])
    mu = jnp.mean(k_raw, axis=-1, keepdims=True)
    var = jnp.mean((k_raw - mu) * (k_raw - mu), axis=-1, keepdims=True)
    qk = (k_raw - mu) / jnp.sqrt(var + 1e-5) * lng_ref[...] + lnb_ref[...]

    qsq = jnp.sum(qk * qk, axis=-1, keepdims=True)
    for c in range(_MP // _MC):
        sl = pl.ds(c * _MC, _MC)
        mkc = mkT_ref[:, sl]
        ksq = jnp.sum(mkc * mkc, axis=0, keepdims=True)
        d2 = (qsq + ksq
              - 2.0 * jnp.dot(qk, mkc, preferred_element_type=jnp.float32))
        col = jax.lax.broadcasted_iota(jnp.int32, (1, _MC), 1) + c * _MC
        d2_ref[:, sl] = jnp.where(col < _M, d2, jnp.inf)

    vals = mv_ref[...]

    def body(_, carry):
        num, den, cnt = carry
        d2c = d2_ref[...]
        m = jnp.min(d2c, axis=1, keepdims=True)
        eq = d2c == m
        c = jnp.sum(eq.astype(jnp.float32), axis=1, keepdims=True)
        vsum = jnp.sum(jnp.where(eq, vals, 0.0), axis=1, keepdims=True)
        d2_ref[...] = jnp.where(eq, jnp.inf, d2c)
        w = 1.0 / (m + 1e-7)
        active = cnt < float(_K)
        num = num + jnp.where(active, w * vsum, 0.0)
        den = den + jnp.where(active, w * c, 0.0)
        cnt = cnt + jnp.where(active, c, 0.0)
        return (num, den, cnt)

    zeros = jnp.zeros((_BQ, 1), jnp.float32)
    num, den, _ = jax.lax.fori_loop(0, _K, body, (zeros, zeros, zeros))
    memory_value = num / den

    hv = jnp.maximum(
        jnp.dot(s, V1_ref[...], preferred_element_type=jnp.float32)
        + bv1_ref[...], 0.0)
    hv2 = jnp.maximum(
        jnp.dot(hv, V2_ref[...], preferred_element_type=jnp.float32)
        + bv2_ref[...], 0.0)
    network_value = (jnp.sum(hv2 * V3_ref[...], axis=-1, keepdims=True)
                     + bv3_ref[...])

    combined = 0.9 * memory_value + 0.1 * network_value
    out_ref[...] = combined


@jax.jit
def kernel(state, W1, b1, W2, b2, ln_g, ln_b, mem_keys, mem_values,
           V1, bv1, V2, bv2, V3, bv3):
    Q = state.shape[0]
    mkT = jnp.pad(mem_keys.T, ((0, 0), (0, _MP - _M)))
    mv = jnp.pad(mem_values, (0, _MP - _M)).reshape(1, _MP)

    b1r = b1.reshape(1, -1)
    b2r = b2.reshape(1, -1)
    lngr = ln_g.reshape(1, -1)
    lnbr = ln_b.reshape(1, -1)
    bv1r = bv1.reshape(1, -1)
    bv2r = bv2.reshape(1, -1)
    V3r = V3.reshape(1, -1)
    bv3r = bv3.reshape(1, -1)

    grid = Q // _BQ
    full = lambda shape: pl.BlockSpec(shape, lambda i: (0, 0))
    out = pl.pallas_call(
        _nec_kernel,
        grid=(grid,),
        in_specs=[
            pl.BlockSpec((_BQ, state.shape[1]), lambda i: (i, 0)),
            full(W1.shape), full(b1r.shape), full(W2.shape), full(b2r.shape),
            full(lngr.shape), full(lnbr.shape),
            full(mkT.shape), full(mv.shape),
            full(V1.shape), full(bv1r.shape), full(V2.shape), full(bv2r.shape),
            full(V3r.shape), full(bv3r.shape),
        ],
        out_specs=pl.BlockSpec((_BQ, 1), lambda i: (i, 0)),
        out_shape=jax.ShapeDtypeStruct((Q, 1), jnp.float32),
        scratch_shapes=[pltpu.VMEM((_BQ, _MP), jnp.float32)],
    )(state, W1, b1r, W2, b2r, lngr, lnbr, mkT, mv,
      V1, bv1r, V2, bv2r, V3r, bv3r)
    return out.reshape(Q)


# speedup vs baseline: 2.2897x; 1.7822x over previous
import functools

import jax
import jax.numpy as jnp
from jax import lax
from jax.experimental import pallas as pl
from jax.experimental.pallas import tpu as pltpu
from jax.experimental.pallas import tpu_sc as plsc

_M = 50000
_MP = 50176
_MC = _MP // 4
_BQ = 32
_K = 50
_Q = 1024
_NW = 32
_RW = _Q // _NW
_CAP = 2048
_SAMP = 1024
_SRANK = 16


def _tc_kernel(state_ref, W1_ref, b1_ref, W2_ref, b2_ref, lng_ref, lnb_ref,
               mkT_ref, V1_ref, bv1_ref, V2_ref, bv2_ref, V3_ref, bv3_ref,
               d2_ref, T_ref, nv_ref):
    s = state_ref[...]

    h = jnp.maximum(
        jnp.dot(s, W1_ref[...], preferred_element_type=jnp.float32)
        + b1_ref[...], 0.0)
    k_raw = (jnp.dot(h, W2_ref[...], preferred_element_type=jnp.float32)
             + b2_ref[...])
    mu = jnp.mean(k_raw, axis=-1, keepdims=True)
    var = jnp.mean((k_raw - mu) * (k_raw - mu), axis=-1, keepdims=True)
    qk = (k_raw - mu) / jnp.sqrt(var + 1e-5) * lng_ref[...] + lnb_ref[...]

    qsq = jnp.sum(qk * qk, axis=-1, keepdims=True)
    samp = None
    for c in range(_MP // _MC):
        sl = pl.ds(c * _MC, _MC)
        mkc = mkT_ref[:, sl]
        ksq = jnp.sum(mkc * mkc, axis=0, keepdims=True)
        d2 = (qsq + ksq
              - 2.0 * jnp.dot(qk, mkc, preferred_element_type=jnp.float32))
        col = jax.lax.broadcasted_iota(jnp.int32, (1, _MC), 1) + c * _MC
        d2 = jnp.where(col < _M, d2, jnp.inf)
        d2_ref[:, sl] = d2
        if c == 0:
            samp = d2[:, :_SAMP]

    T = None
    sm = samp
    for _ in range(_SRANK):
        m = jnp.min(sm, axis=1, keepdims=True)
        sm = jnp.where(sm == m, jnp.inf, sm)
        T = m
    T_ref[...] = T

    hv = jnp.maximum(
        jnp.dot(s, V1_ref[...], preferred_element_type=jnp.float32)
        + bv1_ref[...], 0.0)
    hv2 = jnp.maximum(
        jnp.dot(hv, V2_ref[...], preferred_element_type=jnp.float32)
        + bv2_ref[...], 0.0)
    nv_ref[...] = (jnp.sum(hv2 * V3_ref[...], axis=-1, keepdims=True)
                   + bv3_ref[...])


def _sc_kernel(d2_hbm, T_hbm, nv_hbm, vals_hbm, out_hbm,
               d2row, vals_v, Tloc, nvloc, outloc, cand_d, cand_i):
    wid = lax.axis_index("s") * 2 + lax.axis_index("c")
    base = wid * _RW
    pltpu.sync_copy(vals_hbm, vals_v)
    pltpu.sync_copy(T_hbm.at[pl.ds(base, _RW)], Tloc)
    pltpu.sync_copy(nv_hbm.at[pl.ds(base, _RW)], nvloc)

    iota16 = lax.broadcasted_iota(jnp.int32, (16,), 0)
    inf16 = jnp.full((16,), jnp.inf, jnp.float32)

    def get_scalar(ref, r):
        return plsc.load_gather(ref, [jnp.full((16,), r, jnp.int32)])[0]

    def sdiv(a, b):
        return (jnp.full((16,), a, jnp.float32)
                / jnp.full((16,), b, jnp.float32))[0]

    def select50(ref_d, get_idx, nregs):
        def mpass(i, mv):
            return jnp.minimum(mv, ref_d[pl.ds(i * 16, 16)])
        m0 = jnp.min(lax.fori_loop(0, nregs, mpass, inf16))

        def it(_, carry):
            num, den, cnt, ms = carry

            def ex(i, c2):
                cvec, vvec, nmin = c2
                x = ref_d[pl.ds(i * 16, 16)]
                eq = x == ms
                vv = plsc.load_gather(vals_v, [get_idx(i)])
                cvec = cvec + jnp.where(eq, 1, 0)
                vvec = vvec + jnp.where(eq, vv, 0.0)
                x2 = jnp.where(eq, jnp.inf, x)
                ref_d[pl.ds(i * 16, 16)] = x2
                return cvec, vvec, jnp.minimum(nmin, x2)

            cvec, vvec, nmin = lax.fori_loop(
                0, nregs, ex,
                (jnp.zeros((16,), jnp.int32), jnp.zeros((16,), jnp.float32),
                 inf16))
            cs = jnp.sum(cvec).astype(jnp.float32)
            vs = jnp.sum(vvec)
            w = sdiv(1.0, ms + 1e-7)
            active = cnt < float(_K)
            num = num + jnp.where(active, w * vs, 0.0)
            den = den + jnp.where(active, w * cs, 0.0)
            cnt = cnt + jnp.where(active, cs, 0.0)
            return num, den, cnt, jnp.min(nmin)

        num, den, _, _ = lax.fori_loop(
            0, _K, it,
            (jnp.float32(0.0), jnp.float32(0.0), jnp.float32(0.0), m0))
        return sdiv(num, den)

    def row_body(r, _):
        pltpu.sync_copy(d2_hbm.at[base + r], d2row)
        Tq = get_scalar(Tloc, r)

        def initb(i, _c):
            cand_d[pl.ds(i * 16, 16)] = inf16
            cand_i[pl.ds(i * 16, 16)] = jnp.zeros((16,), jnp.int32)
            return 0
        lax.fori_loop(0, _CAP // 16, initb, 0)

        def coll(c, carry):
            off, ovf = carry
            x = d2row[pl.ds(c * 16, 16)]
            msk = x <= Tq
            n = plsc.all_reduce_population_count(msk)[0]
            ok = (off + 16) <= _CAP

            @pl.when(ok & (n > 0))
            def _():
                plsc.store_compressed(cand_d.at[pl.ds(off, 16)], x, mask=msk)
                plsc.store_compressed(cand_i.at[pl.ds(off, 16)],
                                      iota16 + c * 16, mask=msk)

            off = jnp.where(ok, off + n, off)
            ovf = ovf | (jnp.logical_not(ok) & (n > 0))
            return off, ovf

        off, ovf = lax.fori_loop(0, _MP // 16, coll,
                                 (jnp.int32(0), jnp.bool_(False)))
        valid = jnp.logical_not(ovf) & (off >= _K)

        mv = lax.cond(
            valid,
            lambda: select50(cand_d, lambda i: cand_i[pl.ds(i * 16, 16)],
                             (off + 15) // 16),
            lambda: select50(d2row, lambda i: iota16 + i * 16, _MP // 16))

        combined = 0.9 * mv + 0.1 * get_scalar(nvloc, r)
        b16 = (r // 16) * 16
        ov = outloc[pl.ds(b16, 16)]
        outloc[pl.ds(b16, 16)] = jnp.where(iota16 == (r - b16), combined, ov)
        return 0

    lax.fori_loop(0, _RW, row_body, 0)
    pltpu.sync_copy(outloc, out_hbm.at[pl.ds(base, _RW)])


@jax.jit
def kernel(state, W1, b1, W2, b2, ln_g, ln_b, mem_keys, mem_values,
           V1, bv1, V2, bv2, V3, bv3):
    mkT = jnp.pad(mem_keys.T, ((0, 0), (0, _MP - _M)))
    mv = jnp.pad(mem_values, (0, _MP - _M))

    b1r = b1.reshape(1, -1)
    b2r = b2.reshape(1, -1)
    lngr = ln_g.reshape(1, -1)
    lnbr = ln_b.reshape(1, -1)
    bv1r = bv1.reshape(1, -1)
    bv2r = bv2.reshape(1, -1)
    V3r = V3.reshape(1, -1)
    bv3r = bv3.reshape(1, -1)

    grid = _Q // _BQ
    full = lambda shape: pl.BlockSpec(shape, lambda i: (0, 0))
    d2, T, nv = pl.pallas_call(
        _tc_kernel,
        grid=(grid,),
        in_specs=[
            pl.BlockSpec((_BQ, state.shape[1]), lambda i: (i, 0)),
            full(W1.shape), full(b1r.shape), full(W2.shape), full(b2r.shape),
            full(lngr.shape), full(lnbr.shape), full(mkT.shape),
            full(V1.shape), full(bv1r.shape), full(V2.shape), full(bv2r.shape),
            full(V3r.shape), full(bv3r.shape),
        ],
        out_specs=[
            pl.BlockSpec((_BQ, _MP), lambda i: (i, 0)),
            pl.BlockSpec((_BQ, 1), lambda i: (i, 0)),
            pl.BlockSpec((_BQ, 1), lambda i: (i, 0)),
        ],
        out_shape=[
            jax.ShapeDtypeStruct((_Q, _MP), jnp.float32),
            jax.ShapeDtypeStruct((_Q, 1), jnp.float32),
            jax.ShapeDtypeStruct((_Q, 1), jnp.float32),
        ],
    )(state, W1, b1r, W2, b2r, lngr, lnbr, mkT,
      V1, bv1r, V2, bv2r, V3r, bv3r)

    sc = functools.partial(
        pl.kernel,
        mesh=plsc.VectorSubcoreMesh(core_axis_name="c", subcore_axis_name="s"),
        out_type=jax.ShapeDtypeStruct((_Q,), jnp.float32),
        compiler_params=pltpu.CompilerParams(needs_layout_passes=False),
        scratch_types=[
            pltpu.VMEM((_MP,), jnp.float32),
            pltpu.VMEM((_MP,), jnp.float32),
            pltpu.VMEM((_RW,), jnp.float32),
            pltpu.VMEM((_RW,), jnp.float32),
            pltpu.VMEM((_RW,), jnp.float32),
            pltpu.VMEM((_CAP,), jnp.float32),
            pltpu.VMEM((_CAP,), jnp.int32),
        ],
    )(_sc_kernel)
    out = sc(d2, T.reshape(_Q), nv.reshape(_Q), mv)
    return out
